# baseline (speedup 1.0000x reference)
import functools

import jax
import jax.numpy as jnp
from jax import lax
from jax.experimental import pallas as pl
from jax.experimental.pallas import tpu as pltpu
from jax.experimental.pallas import tpu_sc as plsc

N = 10000
E = 320000
NC, NS = 2, 16
B = 128
NB = 4
EPS = 20480
CHT = EPS // B
CHH = CHT // 2
NP = 10240
ZR = 64
RPT = NP // NS
D2 = 64

_MESH = plsc.VectorSubcoreMesh(
    core_axis_name="c", subcore_axis_name="s", num_cores=NC, num_subcores=NS
)


@functools.partial(
    pl.kernel,
    out_type=jax.ShapeDtypeStruct((NC, NP, 16), jnp.float32),
    mesh=_MESH,
    compiler_params=pltpu.CompilerParams(use_tc_tiling_on_sc=False),
    scratch_types=[
        pltpu.VMEM((CHH, B), jnp.int32),
        pltpu.VMEM((B, 16), jnp.float32),
        pltpu.VMEM((ZR, 16), jnp.float32),
        pltpu.VMEM_SHARED((NP, 16), jnp.float32),
    ],
)
def _sc_degree(dst_hbm, out_hbm, dst_v, ones_v, zbuf, acc):
    c = lax.axis_index("c")
    s = lax.axis_index("s")

    pltpu.sync_copy(dst_hbm.at[s, pl.ds(c * CHH, CHH)], dst_v)

    @pl.loop(0, B)
    def _(i):
        ones_v[i, :] = jnp.ones((16,), jnp.float32)

    @pl.loop(0, ZR)
    def _(i):
        zbuf[i, :] = jnp.zeros((16,), jnp.float32)

    @pl.loop(0, RPT, step=ZR)
    def _(r):
        pltpu.sync_copy(zbuf, acc.at[pl.ds(s * RPT + r, ZR)])

    plsc.subcore_barrier()

    @pl.loop(0, CHH)
    def _(ch):
        pltpu.sync_copy(ones_v, acc.at[dst_v.at[ch]], add=True)

    plsc.subcore_barrier()
    pltpu.sync_copy(acc.at[pl.ds(s * RPT, RPT)],
                    out_hbm.at[c, pl.ds(s * RPT, RPT)])


def _make_sc_gather_scatter(split_cols):
    nch = CHT if split_cols else CHH

    @functools.partial(
        pl.kernel,
        out_type=jax.ShapeDtypeStruct((NC, NP, D2), jnp.float32),
        mesh=_MESH,
        compiler_params=pltpu.CompilerParams(use_tc_tiling_on_sc=False),
        scratch_types=[
            pltpu.VMEM((nch, B), jnp.int32),
            pltpu.VMEM((nch, B), jnp.int32),
            [pltpu.VMEM((B, D2), jnp.float32)] * NB,
            pltpu.VMEM((ZR, D2), jnp.float32),
            pltpu.VMEM_SHARED((NP, D2), jnp.float32),
            [pltpu.SemaphoreType.DMA] * NB,
            [pltpu.SemaphoreType.DMA] * NB,
        ],
    )
    def sc_kernel(src_hbm, dst_hbm, g_hbm, out_hbm,
                  src_v, dst_v, bufs, zbuf, acc, gsems, ssems):
        c = lax.axis_index("c")
        s = lax.axis_index("s")

        if split_cols:
            pltpu.async_copy(src_hbm.at[s], src_v, gsems[0])
            pltpu.async_copy(dst_hbm.at[s], dst_v, gsems[1])
            table = g_hbm.at[c]
        else:
            pltpu.async_copy(src_hbm.at[s, pl.ds(c * CHH, CHH)], src_v,
                             gsems[0])
            pltpu.async_copy(dst_hbm.at[s, pl.ds(c * CHH, CHH)], dst_v,
                             gsems[1])
            table = g_hbm

        @pl.loop(0, ZR)
        def _(i):
            @pl.loop(0, D2, step=16)
            def _(j):
                zbuf[i, pl.ds(j, 16)] = jnp.zeros((16,), jnp.float32)

        @pl.loop(0, RPT, step=ZR)
        def _(r):
            pltpu.sync_copy(zbuf, acc.at[pl.ds(s * RPT + r, ZR)])

        pltpu.make_async_copy(src_hbm.at[s], src_v, gsems[0]).wait()
        pltpu.make_async_copy(dst_hbm.at[s], dst_v, gsems[1]).wait()

        plsc.subcore_barrier()

        for j in range(NB):
            pltpu.async_copy(table.at[src_v.at[j]], bufs[j], gsems[j])

        @pl.loop(0, nch, step=NB)
        def _(ch):
            for j in range(NB):
                pltpu.make_async_copy(table.at[src_v.at[ch + j]],
                                      bufs[j], gsems[j]).wait()

                @pl.when(ch + NB + j < nch)
                def _():
                    pltpu.async_copy(table.at[src_v.at[ch + NB + j]],
                                     bufs[j], gsems[j])

        plsc.subcore_barrier()
        pltpu.sync_copy(acc.at[pl.ds(s * RPT, RPT)],
                        out_hbm.at[c, pl.ds(s * RPT, RPT)])

    return sc_kernel


_sc_gs_cols = _make_sc_gather_scatter(True)
_sc_gs_edges = _make_sc_gather_scatter(False)


@functools.partial(
    pl.kernel,
    out_type=jax.ShapeDtypeStruct((NC, NP, D2), jnp.float32),
    mesh=_MESH,
    compiler_params=pltpu.CompilerParams(use_tc_tiling_on_sc=False),
    scratch_types=[
        pltpu.VMEM((CHT, B), jnp.int32),
        pltpu.VMEM((B, 2 * D2), jnp.float32),
        pltpu.VMEM((B, 2 * D2), jnp.float32),
        pltpu.VMEM_SHARED((NP, D2), jnp.float32),
        pltpu.SemaphoreType.DMA,
        pltpu.SemaphoreType.DMA,
    ],
)
def _sc_diag_full(src_hbm, g_hbm, out_hbm, src_v, buf0, buf1, acc, g0, g1):
    s = lax.axis_index("s")
    pltpu.sync_copy(src_hbm.at[s], src_v)

    pltpu.async_copy(g_hbm.at[src_v.at[0]], buf0, g0)
    pltpu.async_copy(g_hbm.at[src_v.at[1]], buf1, g1)

    @pl.loop(0, CHT, step=2)
    def _(ch):
        pltpu.make_async_copy(g_hbm.at[src_v.at[ch]], buf0, g0).wait()

        @pl.when(ch + 2 < CHT)
        def _():
            pltpu.async_copy(g_hbm.at[src_v.at[ch + 2]], buf0, g0)

        pltpu.make_async_copy(g_hbm.at[src_v.at[ch]], buf1, g1).wait()

        @pl.when(ch + 3 < CHT)
        def _():
            pltpu.async_copy(g_hbm.at[src_v.at[ch + 3]], buf1, g1)

    pltpu.sync_copy(acc.at[pl.ds(s * RPT, RPT)],
                    out_hbm.at[lax.axis_index("c"), pl.ds(s * RPT, RPT)])


_R = 2000

_CNT_SPEC = pl.BlockSpec((NC, _R, 16), lambda i: (0, i, 0))


def _tc_matmul(x, w):
    def body(x_ref, w_ref, o_ref):
        o_ref[...] = jnp.dot(x_ref[...], w_ref[...],
                             preferred_element_type=jnp.float32)

    return pl.pallas_call(
        body,
        grid=(N // _R,),
        in_specs=[
            pl.BlockSpec((_R, x.shape[1]), lambda i: (i, 0)),
            pl.BlockSpec(w.shape, lambda i: (0, 0)),
        ],
        out_specs=pl.BlockSpec((_R, w.shape[1]), lambda i: (i, 0)),
        out_shape=jax.ShapeDtypeStruct((N, w.shape[1]), jnp.float32),
    )(x, w)


def _tc_scale(cnt, h):
    def body(cnt_ref, h_ref, o_ref):
        dinv = lax.rsqrt(1.0 + cnt_ref[0, :, 0:1] + cnt_ref[1, :, 0:1])
        g = h_ref[...] * dinv
        o_ref[0] = g[:, :D2]
        o_ref[1] = g[:, D2:]

    return pl.pallas_call(
        body,
        grid=(N // _R,),
        in_specs=[
            _CNT_SPEC,
            pl.BlockSpec((_R, 2 * D2), lambda i: (i, 0)),
        ],
        out_specs=pl.BlockSpec((NC, _R, D2), lambda i: (0, i, 0)),
        out_shape=jax.ShapeDtypeStruct((NC, N, D2), jnp.float32),
    )(cnt, h)


def _tc_combine12(cnt, acc, g, b, w):
    def body(cnt_ref, acc_ref, g_ref, b_ref, w_ref, o_ref):
        dinv = lax.rsqrt(1.0 + cnt_ref[0, :, 0:1] + cnt_ref[1, :, 0:1])
        t = jnp.concatenate(
            [acc_ref[0] + g_ref[0], acc_ref[1] + g_ref[1]], axis=1)
        t = t * dinv + jnp.reshape(b_ref[...], (1, 2 * D2))
        t = jnp.maximum(t, 0.0)
        r = jnp.dot(t, w_ref[...], preferred_element_type=jnp.float32) * dinv
        o_ref[0] = r[:, :D2]
        o_ref[1] = r[:, D2:]

    return pl.pallas_call(
        body,
        grid=(N // _R,),
        in_specs=[
            _CNT_SPEC,
            pl.BlockSpec((NC, _R, D2), lambda i: (0, i, 0)),
            pl.BlockSpec((NC, _R, D2), lambda i: (0, i, 0)),
            pl.BlockSpec((2 * D2,), lambda i: (0,)),
            pl.BlockSpec((2 * D2, 2 * D2), lambda i: (0, 0)),
        ],
        out_specs=pl.BlockSpec((NC, _R, D2), lambda i: (0, i, 0)),
        out_shape=jax.ShapeDtypeStruct((NC, N, D2), jnp.float32),
    )(cnt, acc, g, b, w)


def _tc_combine23(cnt, acc, g, b, w):
    def body(cnt_ref, acc_ref, g_ref, b_ref, w_ref, o_ref):
        dinv = lax.rsqrt(1.0 + cnt_ref[0, :, 0:1] + cnt_ref[1, :, 0:1])
        t = jnp.concatenate(
            [acc_ref[0] + g_ref[0], acc_ref[1] + g_ref[1]], axis=1)
        t = t * dinv + jnp.reshape(b_ref[...], (1, 2 * D2))
        t = jnp.maximum(t, 0.0)
        o_ref[...] = jnp.dot(t, w_ref[...],
                             preferred_element_type=jnp.float32) * dinv

    return pl.pallas_call(
        body,
        grid=(N // _R,),
        in_specs=[
            _CNT_SPEC,
            pl.BlockSpec((NC, _R, D2), lambda i: (0, i, 0)),
            pl.BlockSpec((NC, _R, D2), lambda i: (0, i, 0)),
            pl.BlockSpec((2 * D2,), lambda i: (0,)),
            pl.BlockSpec((2 * D2, D2), lambda i: (0, 0)),
        ],
        out_specs=pl.BlockSpec((_R, D2), lambda i: (i, 0)),
        out_shape=jax.ShapeDtypeStruct((N, D2), jnp.float32),
    )(cnt, acc, g, b, w)


def _tc_final(cnt, acc, g, b):
    def body(cnt_ref, acc_ref, g_ref, b_ref, o_ref):
        dinv = lax.rsqrt(1.0 + cnt_ref[0, :, 0:1] + cnt_ref[1, :, 0:1])
        t = (acc_ref[0] + acc_ref[1] + g_ref[...]) * dinv
        o_ref[...] = t + jnp.reshape(b_ref[...], (1, D2))

    return pl.pallas_call(
        body,
        grid=(N // _R,),
        in_specs=[
            _CNT_SPEC,
            pl.BlockSpec((NC, _R, D2), lambda i: (0, i, 0)),
            pl.BlockSpec((_R, D2), lambda i: (i, 0)),
            pl.BlockSpec((D2,), lambda i: (0,)),
        ],
        out_specs=pl.BlockSpec((_R, D2), lambda i: (i, 0)),
        out_shape=jax.ShapeDtypeStruct((N, D2), jnp.float32),
    )(cnt, acc, g, b)


@jax.jit
def kernel(x, edge_index, W1, b1, W2, b2, W3, b3):
    pad = NS * EPS - E
    src = jnp.concatenate(
        [edge_index[0], jnp.zeros((pad,), jnp.int32)]).reshape(NS, CHT, B)
    dst = jnp.concatenate(
        [edge_index[1], jnp.full((pad,), N, jnp.int32)]).reshape(NS, CHT, B)

    cnt = _sc_degree(dst)

    h1 = _tc_matmul(x, W1)
    g1 = _tc_scale(cnt, h1)

    acc1 = _sc_diag_full(src, h1)
    g2 = _tc_combine12(cnt, acc1, g1, b1, W2)

    acc2 = _sc_gs_cols(src, dst, g2)
    g3 = _tc_combine23(cnt, acc2, g2, b2, W3)

    acc3 = _sc_gs_edges(src, dst, g3)
    return _tc_final(cnt, acc3, g3, b3)


# speedup vs baseline: 1.5235x; 1.5235x over previous
import functools

import jax
import jax.numpy as jnp
from jax import lax
from jax.experimental import pallas as pl
from jax.experimental.pallas import tpu as pltpu
from jax.experimental.pallas import tpu_sc as plsc

N = 10000
E = 320000
NC, NS = 2, 16
B = 128
NB = 8
EPS = 20480
CHT = EPS // B
CHH = CHT // 2
NP = 10240
ZR = 64
RPT = NP // NS
D2 = 64

_MESH = plsc.VectorSubcoreMesh(
    core_axis_name="c", subcore_axis_name="s", num_cores=NC, num_subcores=NS
)


@functools.partial(
    pl.kernel,
    out_type=jax.ShapeDtypeStruct((NC, NP, 16), jnp.float32),
    mesh=_MESH,
    compiler_params=pltpu.CompilerParams(use_tc_tiling_on_sc=False),
    scratch_types=[
        pltpu.VMEM((CHH, B), jnp.int32),
        pltpu.VMEM((B, 16), jnp.float32),
        pltpu.VMEM((ZR, 16), jnp.float32),
        pltpu.VMEM_SHARED((NP, 16), jnp.float32),
    ],
)
def _sc_degree(dst_hbm, out_hbm, dst_v, ones_v, zbuf, acc):
    c = lax.axis_index("c")
    s = lax.axis_index("s")

    pltpu.sync_copy(dst_hbm.at[s, pl.ds(c * CHH, CHH)], dst_v)

    @pl.loop(0, B)
    def _(i):
        ones_v[i, :] = jnp.ones((16,), jnp.float32)

    @pl.loop(0, ZR)
    def _(i):
        zbuf[i, :] = jnp.zeros((16,), jnp.float32)

    @pl.loop(0, RPT, step=ZR)
    def _(r):
        pltpu.sync_copy(zbuf, acc.at[pl.ds(s * RPT + r, ZR)])

    plsc.subcore_barrier()

    @pl.loop(0, CHH)
    def _(ch):
        pltpu.sync_copy(ones_v, acc.at[dst_v.at[ch]], add=True)

    plsc.subcore_barrier()
    pltpu.sync_copy(acc.at[pl.ds(s * RPT, RPT)],
                    out_hbm.at[c, pl.ds(s * RPT, RPT)])


def _make_sc_gather_scatter(split_cols):
    nch = CHT if split_cols else CHH
    nblk = nch // NB

    @functools.partial(
        pl.kernel,
        out_type=jax.ShapeDtypeStruct((NC, NP, D2), jnp.float32),
        mesh=_MESH,
        compiler_params=pltpu.CompilerParams(use_tc_tiling_on_sc=False),
        scratch_types=[
            [pltpu.VMEM((NB, B), jnp.int32)] * 2,
            [pltpu.VMEM((NB, B), jnp.int32)] * 2,
            [pltpu.VMEM((B, D2), jnp.float32)] * NB,
            pltpu.VMEM((ZR, D2), jnp.float32),
            pltpu.VMEM_SHARED((NP, D2), jnp.float32),
            [pltpu.SemaphoreType.DMA] * NB,
            [pltpu.SemaphoreType.DMA] * NB,
            [pltpu.SemaphoreType.DMA] * 2,
        ],
    )
    def sc_kernel(src_hbm, dst_hbm, g_hbm, out_hbm,
                  sstr, dstr, bufs, zbuf, acc, gsems, ssems, isems):
        c = lax.axis_index("c")
        s = lax.axis_index("s")

        if split_cols:
            off = 0
            table = g_hbm.at[c]
        else:
            off = c * CHH
            table = g_hbm

        def load_strip(p, blk):
            pltpu.async_copy(src_hbm.at[s, pl.ds(off + blk * NB, NB)],
                             sstr[p], isems[p])
            pltpu.async_copy(dst_hbm.at[s, pl.ds(off + blk * NB, NB)],
                             dstr[p], isems[p])

        def wait_strip(p, blk):
            pltpu.make_async_copy(src_hbm.at[s, pl.ds(off + blk * NB, NB)],
                                  sstr[p], isems[p]).wait()
            pltpu.make_async_copy(dst_hbm.at[s, pl.ds(off + blk * NB, NB)],
                                  dstr[p], isems[p]).wait()

        load_strip(0, 0)
        load_strip(1, 1)

        @pl.loop(0, ZR)
        def _(i):
            @pl.loop(0, D2, step=16)
            def _(j):
                zbuf[i, pl.ds(j, 16)] = jnp.zeros((16,), jnp.float32)

        @pl.loop(0, RPT, step=ZR)
        def _(r):
            pltpu.sync_copy(zbuf, acc.at[pl.ds(s * RPT + r, ZR)])

        plsc.subcore_barrier()

        wait_strip(0, 0)
        for j in range(NB):
            pltpu.async_copy(table.at[sstr[0].at[j]], bufs[j], gsems[j])

        def half(k, p):
            q = 1 - p
            for j in range(NB):
                pltpu.make_async_copy(table.at[sstr[p].at[j]],
                                      bufs[j], gsems[j]).wait()
                pltpu.async_copy(bufs[j], acc.at[dstr[p].at[j]],
                                 ssems[j], add=True)

            @pl.when(k + 1 < nblk)
            def _():
                wait_strip(q, k + 1)

            for j in range(NB):
                pltpu.make_async_copy(bufs[j], acc.at[dstr[p].at[j]],
                                      ssems[j]).wait()

                @pl.when(k + 1 < nblk)
                def _():
                    pltpu.async_copy(table.at[sstr[q].at[j]],
                                     bufs[j], gsems[j])

            @pl.when(k + 2 < nblk)
            def _():
                load_strip(p, k + 2)

        @pl.loop(0, nblk, step=2)
        def _(k):
            half(k, 0)
            half(k + 1, 1)

        plsc.subcore_barrier()
        pltpu.sync_copy(acc.at[pl.ds(s * RPT, RPT)],
                        out_hbm.at[c, pl.ds(s * RPT, RPT)])

    return sc_kernel


_sc_gs_cols = _make_sc_gather_scatter(True)
_sc_gs_edges = _make_sc_gather_scatter(False)


_R = 2000

_CNT_SPEC = pl.BlockSpec((NC, _R, 16), lambda i: (0, i, 0))


def _tc_matmul(x, w):
    def body(x_ref, w_ref, o_ref):
        o_ref[...] = jnp.dot(x_ref[...], w_ref[...],
                             preferred_element_type=jnp.float32)

    return pl.pallas_call(
        body,
        grid=(N // _R,),
        in_specs=[
            pl.BlockSpec((_R, x.shape[1]), lambda i: (i, 0)),
            pl.BlockSpec(w.shape, lambda i: (0, 0)),
        ],
        out_specs=pl.BlockSpec((_R, w.shape[1]), lambda i: (i, 0)),
        out_shape=jax.ShapeDtypeStruct((N, w.shape[1]), jnp.float32),
    )(x, w)


def _tc_scale(cnt, h):
    def body(cnt_ref, h_ref, o_ref):
        dinv = lax.rsqrt(1.0 + cnt_ref[0, :, 0:1] + cnt_ref[1, :, 0:1])
        g = h_ref[...] * dinv
        o_ref[0] = g[:, :D2]
        o_ref[1] = g[:, D2:]

    return pl.pallas_call(
        body,
        grid=(N // _R,),
        in_specs=[
            _CNT_SPEC,
            pl.BlockSpec((_R, 2 * D2), lambda i: (i, 0)),
        ],
        out_specs=pl.BlockSpec((NC, _R, D2), lambda i: (0, i, 0)),
        out_shape=jax.ShapeDtypeStruct((NC, N, D2), jnp.float32),
    )(cnt, h)


def _tc_combine12(cnt, acc, g, b, w):
    def body(cnt_ref, acc_ref, g_ref, b_ref, w_ref, o_ref):
        dinv = lax.rsqrt(1.0 + cnt_ref[0, :, 0:1] + cnt_ref[1, :, 0:1])
        t = jnp.concatenate(
            [acc_ref[0] + g_ref[0], acc_ref[1] + g_ref[1]], axis=1)
        t = t * dinv + jnp.reshape(b_ref[...], (1, 2 * D2))
        t = jnp.maximum(t, 0.0)
        r = jnp.dot(t, w_ref[...], preferred_element_type=jnp.float32) * dinv
        o_ref[0] = r[:, :D2]
        o_ref[1] = r[:, D2:]

    return pl.pallas_call(
        body,
        grid=(N // _R,),
        in_specs=[
            _CNT_SPEC,
            pl.BlockSpec((NC, _R, D2), lambda i: (0, i, 0)),
            pl.BlockSpec((NC, _R, D2), lambda i: (0, i, 0)),
            pl.BlockSpec((2 * D2,), lambda i: (0,)),
            pl.BlockSpec((2 * D2, 2 * D2), lambda i: (0, 0)),
        ],
        out_specs=pl.BlockSpec((NC, _R, D2), lambda i: (0, i, 0)),
        out_shape=jax.ShapeDtypeStruct((NC, N, D2), jnp.float32),
    )(cnt, acc, g, b, w)


def _tc_combine23(cnt, acc, g, b, w):
    def body(cnt_ref, acc_ref, g_ref, b_ref, w_ref, o_ref):
        dinv = lax.rsqrt(1.0 + cnt_ref[0, :, 0:1] + cnt_ref[1, :, 0:1])
        t = jnp.concatenate(
            [acc_ref[0] + g_ref[0], acc_ref[1] + g_ref[1]], axis=1)
        t = t * dinv + jnp.reshape(b_ref[...], (1, 2 * D2))
        t = jnp.maximum(t, 0.0)
        o_ref[...] = jnp.dot(t, w_ref[...],
                             preferred_element_type=jnp.float32) * dinv

    return pl.pallas_call(
        body,
        grid=(N // _R,),
        in_specs=[
            _CNT_SPEC,
            pl.BlockSpec((NC, _R, D2), lambda i: (0, i, 0)),
            pl.BlockSpec((NC, _R, D2), lambda i: (0, i, 0)),
            pl.BlockSpec((2 * D2,), lambda i: (0,)),
            pl.BlockSpec((2 * D2, D2), lambda i: (0, 0)),
        ],
        out_specs=pl.BlockSpec((_R, D2), lambda i: (i, 0)),
        out_shape=jax.ShapeDtypeStruct((N, D2), jnp.float32),
    )(cnt, acc, g, b, w)


def _tc_final(cnt, acc, g, b):
    def body(cnt_ref, acc_ref, g_ref, b_ref, o_ref):
        dinv = lax.rsqrt(1.0 + cnt_ref[0, :, 0:1] + cnt_ref[1, :, 0:1])
        t = (acc_ref[0] + acc_ref[1] + g_ref[...]) * dinv
        o_ref[...] = t + jnp.reshape(b_ref[...], (1, D2))

    return pl.pallas_call(
        body,
        grid=(N // _R,),
        in_specs=[
            _CNT_SPEC,
            pl.BlockSpec((NC, _R, D2), lambda i: (0, i, 0)),
            pl.BlockSpec((_R, D2), lambda i: (i, 0)),
            pl.BlockSpec((D2,), lambda i: (0,)),
        ],
        out_specs=pl.BlockSpec((_R, D2), lambda i: (i, 0)),
        out_shape=jax.ShapeDtypeStruct((N, D2), jnp.float32),
    )(cnt, acc, g, b)


@jax.jit
def kernel(x, edge_index, W1, b1, W2, b2, W3, b3):
    pad = NS * EPS - E
    src = jnp.concatenate(
        [edge_index[0], jnp.zeros((pad,), jnp.int32)]).reshape(NS, CHT, B)
    dst = jnp.concatenate(
        [edge_index[1], jnp.full((pad,), N, jnp.int32)]).reshape(NS, CHT, B)

    cnt = _sc_degree(dst)

    h1 = _tc_matmul(x, W1)
    g1 = _tc_scale(cnt, h1)

    acc1 = _sc_gs_cols(src, dst, g1)
    g2 = _tc_combine12(cnt, acc1, g1, b1, W2)

    acc2 = _sc_gs_cols(src, dst, g2)
    g3 = _tc_combine23(cnt, acc2, g2, b2, W3)

    acc3 = _sc_gs_edges(src, dst, g3)
    return _tc_final(cnt, acc3, g3, b3)


# speedup vs baseline: 2.2924x; 1.5047x over previous
import functools

import jax
import jax.numpy as jnp
from jax import lax
from jax.experimental import pallas as pl
from jax.experimental.pallas import tpu as pltpu
from jax.experimental.pallas import tpu_sc as plsc

N = 10000
E = 320000
NC, NS = 2, 16
B = 128
NB = 8
EPS = 20480
CHT = EPS // B
CHH = CHT // 2
NP = 10240
ZR = 64
RPT = NP // NS
D2 = 64

_MESH = plsc.VectorSubcoreMesh(
    core_axis_name="c", subcore_axis_name="s", num_cores=NC, num_subcores=NS
)


@functools.partial(
    pl.kernel,
    out_type=jax.ShapeDtypeStruct((NC, NP, 16), jnp.float32),
    mesh=_MESH,
    compiler_params=pltpu.CompilerParams(use_tc_tiling_on_sc=False),
    scratch_types=[
        pltpu.VMEM((CHH, B), jnp.int32),
        pltpu.VMEM((B, 16), jnp.float32),
        pltpu.VMEM((ZR, 16), jnp.float32),
        pltpu.VMEM_SHARED((NP, 16), jnp.float32),
    ],
)
def _sc_degree(dst_hbm, out_hbm, dst_v, ones_v, zbuf, acc):
    c = lax.axis_index("c")
    s = lax.axis_index("s")

    pltpu.sync_copy(dst_hbm.at[s, pl.ds(c * CHH, CHH)], dst_v)

    @pl.loop(0, B)
    def _(i):
        ones_v[i, :] = jnp.ones((16,), jnp.float32)

    @pl.loop(0, ZR)
    def _(i):
        zbuf[i, :] = jnp.zeros((16,), jnp.float32)

    @pl.loop(0, RPT, step=ZR)
    def _(r):
        pltpu.sync_copy(zbuf, acc.at[pl.ds(s * RPT + r, ZR)])

    plsc.subcore_barrier()

    @pl.loop(0, CHH)
    def _(ch):
        pltpu.sync_copy(ones_v, acc.at[dst_v.at[ch]], add=True)

    plsc.subcore_barrier()
    pltpu.sync_copy(acc.at[pl.ds(s * RPT, RPT)],
                    out_hbm.at[c, pl.ds(s * RPT, RPT)])


def _make_sc_gather_scatter(split_cols):
    nch = CHT if split_cols else CHH
    nblk = nch // NB

    @functools.partial(
        pl.kernel,
        out_type=jax.ShapeDtypeStruct((NC, NP, D2), jnp.bfloat16),
        mesh=_MESH,
        compiler_params=pltpu.CompilerParams(use_tc_tiling_on_sc=False),
        scratch_types=[
            [pltpu.VMEM((NB, B), jnp.int32)] * 2,
            [pltpu.VMEM((NB, B), jnp.int32)] * 2,
            [pltpu.VMEM((B, D2), jnp.bfloat16)] * NB,
            pltpu.VMEM((ZR, D2), jnp.bfloat16),
            pltpu.VMEM_SHARED((NP, D2), jnp.bfloat16),
            [pltpu.SemaphoreType.DMA] * NB,
            [pltpu.SemaphoreType.DMA] * NB,
            [pltpu.SemaphoreType.DMA] * 2,
        ],
    )
    def sc_kernel(src_hbm, dst_hbm, g_hbm, out_hbm,
                  sstr, dstr, bufs, zbuf, acc, gsems, ssems, isems):
        c = lax.axis_index("c")
        s = lax.axis_index("s")

        if split_cols:
            off = 0
            table = g_hbm.at[c]
        else:
            off = c * CHH
            table = g_hbm

        def load_strip(p, blk):
            pltpu.async_copy(src_hbm.at[s, pl.ds(off + blk * NB, NB)],
                             sstr[p], isems[p])
            pltpu.async_copy(dst_hbm.at[s, pl.ds(off + blk * NB, NB)],
                             dstr[p], isems[p])

        def wait_strip(p, blk):
            pltpu.make_async_copy(src_hbm.at[s, pl.ds(off + blk * NB, NB)],
                                  sstr[p], isems[p]).wait()
            pltpu.make_async_copy(dst_hbm.at[s, pl.ds(off + blk * NB, NB)],
                                  dstr[p], isems[p]).wait()

        load_strip(0, 0)
        load_strip(1, 1)

        @pl.loop(0, ZR)
        def _(i):
            @pl.loop(0, D2, step=32)
            def _(j):
                zbuf[i, pl.ds(j, 32)] = jnp.zeros((32,), jnp.bfloat16)

        @pl.loop(0, RPT, step=ZR)
        def _(r):
            pltpu.sync_copy(zbuf, acc.at[pl.ds(s * RPT + r, ZR)])

        plsc.subcore_barrier()

        wait_strip(0, 0)
        for j in range(NB):
            pltpu.async_copy(table.at[sstr[0].at[j]], bufs[j], gsems[j])

        def half(k, p):
            q = 1 - p
            for j in range(NB):
                pltpu.make_async_copy(table.at[sstr[p].at[j]],
                                      bufs[j], gsems[j]).wait()
                pltpu.async_copy(bufs[j], acc.at[dstr[p].at[j]],
                                 ssems[j], add=True)

            @pl.when(k + 1 < nblk)
            def _():
                wait_strip(q, k + 1)

            for j in range(NB):
                pltpu.make_async_copy(bufs[j], acc.at[dstr[p].at[j]],
                                      ssems[j]).wait()

                @pl.when(k + 1 < nblk)
                def _():
                    pltpu.async_copy(table.at[sstr[q].at[j]],
                                     bufs[j], gsems[j])

            @pl.when(k + 2 < nblk)
            def _():
                load_strip(p, k + 2)

        @pl.loop(0, nblk, step=2)
        def _(k):
            half(k, 0)
            half(k + 1, 1)

        plsc.subcore_barrier()
        pltpu.sync_copy(acc.at[pl.ds(s * RPT, RPT)],
                        out_hbm.at[c, pl.ds(s * RPT, RPT)])

    return sc_kernel


_sc_gs_cols = _make_sc_gather_scatter(True)
_sc_gs_edges = _make_sc_gather_scatter(False)


_R = 2000

_CNT_SPEC = pl.BlockSpec((NC, _R, 16), lambda i: (0, i, 0))


def _tc_matmul(x, w):
    def body(x_ref, w_ref, o_ref):
        o_ref[...] = jnp.dot(x_ref[...], w_ref[...],
                             preferred_element_type=jnp.float32)

    return pl.pallas_call(
        body,
        grid=(N // _R,),
        in_specs=[
            pl.BlockSpec((_R, x.shape[1]), lambda i: (i, 0)),
            pl.BlockSpec(w.shape, lambda i: (0, 0)),
        ],
        out_specs=pl.BlockSpec((_R, w.shape[1]), lambda i: (i, 0)),
        out_shape=jax.ShapeDtypeStruct((N, w.shape[1]), jnp.float32),
    )(x, w)


def _tc_scale(cnt, h):
    def body(cnt_ref, h_ref, g_ref, t_ref):
        dinv = lax.rsqrt(1.0 + cnt_ref[0, :, 0:1] + cnt_ref[1, :, 0:1])
        g = h_ref[...] * dinv
        g_ref[...] = g
        t_ref[0] = g[:, :D2].astype(jnp.bfloat16)
        t_ref[1] = g[:, D2:].astype(jnp.bfloat16)

    return pl.pallas_call(
        body,
        grid=(N // _R,),
        in_specs=[
            _CNT_SPEC,
            pl.BlockSpec((_R, 2 * D2), lambda i: (i, 0)),
        ],
        out_specs=[
            pl.BlockSpec((_R, 2 * D2), lambda i: (i, 0)),
            pl.BlockSpec((NC, _R, D2), lambda i: (0, i, 0)),
        ],
        out_shape=[
            jax.ShapeDtypeStruct((N, 2 * D2), jnp.float32),
            jax.ShapeDtypeStruct((NC, N, D2), jnp.bfloat16),
        ],
    )(cnt, h)


def _tc_combine12(cnt, acc, g, b, w):
    def body(cnt_ref, acc_ref, g_ref, b_ref, w_ref, o_ref, t_ref):
        dinv = lax.rsqrt(1.0 + cnt_ref[0, :, 0:1] + cnt_ref[1, :, 0:1])
        a = jnp.concatenate([acc_ref[0], acc_ref[1]], axis=1).astype(jnp.float32)
        t = (a + g_ref[...]) * dinv + jnp.reshape(b_ref[...], (1, 2 * D2))
        t = jnp.maximum(t, 0.0)
        r = jnp.dot(t, w_ref[...], preferred_element_type=jnp.float32) * dinv
        o_ref[...] = r
        t_ref[0] = r[:, :D2].astype(jnp.bfloat16)
        t_ref[1] = r[:, D2:].astype(jnp.bfloat16)

    return pl.pallas_call(
        body,
        grid=(N // _R,),
        in_specs=[
            _CNT_SPEC,
            pl.BlockSpec((NC, _R, D2), lambda i: (0, i, 0)),
            pl.BlockSpec((_R, 2 * D2), lambda i: (i, 0)),
            pl.BlockSpec((2 * D2,), lambda i: (0,)),
            pl.BlockSpec((2 * D2, 2 * D2), lambda i: (0, 0)),
        ],
        out_specs=[
            pl.BlockSpec((_R, 2 * D2), lambda i: (i, 0)),
            pl.BlockSpec((NC, _R, D2), lambda i: (0, i, 0)),
        ],
        out_shape=[
            jax.ShapeDtypeStruct((N, 2 * D2), jnp.float32),
            jax.ShapeDtypeStruct((NC, N, D2), jnp.bfloat16),
        ],
    )(cnt, acc, g, b, w)


def _tc_combine23(cnt, acc, g, b, w):
    def body(cnt_ref, acc_ref, g_ref, b_ref, w_ref, o_ref, t_ref):
        dinv = lax.rsqrt(1.0 + cnt_ref[0, :, 0:1] + cnt_ref[1, :, 0:1])
        a = jnp.concatenate([acc_ref[0], acc_ref[1]], axis=1).astype(jnp.float32)
        t = (a + g_ref[...]) * dinv + jnp.reshape(b_ref[...], (1, 2 * D2))
        t = jnp.maximum(t, 0.0)
        r = jnp.dot(t, w_ref[...], preferred_element_type=jnp.float32) * dinv
        o_ref[...] = r
        t_ref[...] = r.astype(jnp.bfloat16)

    return pl.pallas_call(
        body,
        grid=(N // _R,),
        in_specs=[
            _CNT_SPEC,
            pl.BlockSpec((NC, _R, D2), lambda i: (0, i, 0)),
            pl.BlockSpec((_R, 2 * D2), lambda i: (i, 0)),
            pl.BlockSpec((2 * D2,), lambda i: (0,)),
            pl.BlockSpec((2 * D2, D2), lambda i: (0, 0)),
        ],
        out_specs=[
            pl.BlockSpec((_R, D2), lambda i: (i, 0)),
            pl.BlockSpec((_R, D2), lambda i: (i, 0)),
        ],
        out_shape=[
            jax.ShapeDtypeStruct((N, D2), jnp.float32),
            jax.ShapeDtypeStruct((N, D2), jnp.bfloat16),
        ],
    )(cnt, acc, g, b, w)


def _tc_final(cnt, acc, g, b):
    def body(cnt_ref, acc_ref, g_ref, b_ref, o_ref):
        dinv = lax.rsqrt(1.0 + cnt_ref[0, :, 0:1] + cnt_ref[1, :, 0:1])
        a = acc_ref[0].astype(jnp.float32) + acc_ref[1].astype(jnp.float32)
        t = (a + g_ref[...]) * dinv
        o_ref[...] = t + jnp.reshape(b_ref[...], (1, D2))

    return pl.pallas_call(
        body,
        grid=(N // _R,),
        in_specs=[
            _CNT_SPEC,
            pl.BlockSpec((NC, _R, D2), lambda i: (0, i, 0)),
            pl.BlockSpec((_R, D2), lambda i: (i, 0)),
            pl.BlockSpec((D2,), lambda i: (0,)),
        ],
        out_specs=pl.BlockSpec((_R, D2), lambda i: (i, 0)),
        out_shape=jax.ShapeDtypeStruct((N, D2), jnp.float32),
    )(cnt, acc, g, b)


@jax.jit
def kernel(x, edge_index, W1, b1, W2, b2, W3, b3):
    pad = NS * EPS - E
    src = jnp.concatenate(
        [edge_index[0], jnp.zeros((pad,), jnp.int32)]).reshape(NS, CHT, B)
    dst = jnp.concatenate(
        [edge_index[1], jnp.full((pad,), N, jnp.int32)]).reshape(NS, CHT, B)

    cnt = _sc_degree(dst)

    h1 = _tc_matmul(x, W1)
    g1, g1t = _tc_scale(cnt, h1)

    acc1 = _sc_gs_cols(src, dst, g1t)
    g2, g2t = _tc_combine12(cnt, acc1, g1, b1, W2)

    acc2 = _sc_gs_cols(src, dst, g2t)
    g3, g3t = _tc_combine23(cnt, acc2, g2, b2, W3)

    acc3 = _sc_gs_edges(src, dst, g3t)
    return _tc_final(cnt, acc3, g3, b3)


# speedup vs baseline: 4.1087x; 1.7923x over previous
import functools

import jax
import jax.numpy as jnp
from jax import lax
from jax.experimental import pallas as pl
from jax.experimental.pallas import tpu as pltpu
from jax.experimental.pallas import tpu_sc as plsc

N = 10000
E = 320000
NC, NS = 2, 16
B = 128
NB = 8
EPS = 20480
CHT = EPS // B
CHH = CHT // 2
NP = 10240
ZR = 64
RPT = NP // NS
D2 = 64

_MESH = plsc.VectorSubcoreMesh(
    core_axis_name="c", subcore_axis_name="s", num_cores=NC, num_subcores=NS
)


@functools.partial(
    pl.kernel,
    out_type=jax.ShapeDtypeStruct((NC, NP, 16), jnp.float32),
    mesh=_MESH,
    compiler_params=pltpu.CompilerParams(use_tc_tiling_on_sc=False),
    scratch_types=[
        pltpu.VMEM((CHH, B), jnp.int32),
        pltpu.VMEM((B, 16), jnp.float32),
        pltpu.VMEM((ZR, 16), jnp.float32),
        pltpu.VMEM_SHARED((NP, 16), jnp.float32),
    ],
)
def _sc_degree(dst_hbm, out_hbm, dst_v, ones_v, zbuf, acc):
    c = lax.axis_index("c")
    s = lax.axis_index("s")

    pltpu.sync_copy(dst_hbm.at[s, pl.ds(c * CHH, CHH)], dst_v)

    @pl.loop(0, B)
    def _(i):
        ones_v[i, :] = jnp.ones((16,), jnp.float32)

    @pl.loop(0, ZR)
    def _(i):
        zbuf[i, :] = jnp.zeros((16,), jnp.float32)

    @pl.loop(0, RPT, step=ZR)
    def _(r):
        pltpu.sync_copy(zbuf, acc.at[pl.ds(s * RPT + r, ZR)])

    plsc.subcore_barrier()

    @pl.loop(0, CHH)
    def _(ch):
        pltpu.sync_copy(ones_v, acc.at[dst_v.at[ch]], add=True)

    plsc.subcore_barrier()
    pltpu.sync_copy(acc.at[pl.ds(s * RPT, RPT)],
                    out_hbm.at[c, pl.ds(s * RPT, RPT)])


def _make_sc_gather_scatter(split_cols):
    nch = CHT if split_cols else CHH
    nblk = nch // NB

    @functools.partial(
        pl.kernel,
        out_type=jax.ShapeDtypeStruct((NC, NP, D2), jnp.bfloat16),
        mesh=_MESH,
        compiler_params=pltpu.CompilerParams(use_tc_tiling_on_sc=False),
        scratch_types=[
            [pltpu.VMEM((NB, B), jnp.int32)] * 2,
            [pltpu.VMEM((NB, B), jnp.int32)] * 2,
            [pltpu.VMEM((B, D2), jnp.bfloat16)] * NB,
            pltpu.VMEM((ZR, D2), jnp.bfloat16),
            pltpu.VMEM_SHARED((NP, D2), jnp.bfloat16),
            [pltpu.SemaphoreType.DMA] * NB,
            [pltpu.SemaphoreType.DMA] * NB,
            [pltpu.SemaphoreType.DMA] * 2,
        ],
    )
    def sc_kernel(src_hbm, dst_hbm, g_hbm, out_hbm,
                  sstr, dstr, bufs, zbuf, acc, gsems, ssems, isems):
        c = lax.axis_index("c")
        s = lax.axis_index("s")

        if split_cols:
            off = 0
            table = g_hbm.at[c]
        else:
            off = c * CHH
            table = g_hbm

        def load_strip(p, blk):
            pltpu.async_copy(src_hbm.at[s, pl.ds(off + blk * NB, NB)],
                             sstr[p], isems[p])
            pltpu.async_copy(dst_hbm.at[s, pl.ds(off + blk * NB, NB)],
                             dstr[p], isems[p])

        def wait_strip(p, blk):
            pltpu.make_async_copy(src_hbm.at[s, pl.ds(off + blk * NB, NB)],
                                  sstr[p], isems[p]).wait()
            pltpu.make_async_copy(dst_hbm.at[s, pl.ds(off + blk * NB, NB)],
                                  dstr[p], isems[p]).wait()

        load_strip(0, 0)
        load_strip(1, 1)

        @pl.loop(0, ZR)
        def _(i):
            @pl.loop(0, D2, step=32)
            def _(j):
                zbuf[i, pl.ds(j, 32)] = jnp.zeros((32,), jnp.bfloat16)

        @pl.loop(0, RPT, step=ZR)
        def _(r):
            pltpu.sync_copy(zbuf, acc.at[pl.ds(s * RPT + r, ZR)])

        plsc.subcore_barrier()

        wait_strip(0, 0)
        for j in range(NB):
            pltpu.async_copy(table.at[sstr[0].at[j]], bufs[j], gsems[j])

        def half(k, p):
            q = 1 - p
            for j in range(NB):
                pltpu.make_async_copy(table.at[sstr[p].at[j]],
                                      bufs[j], gsems[j]).wait()
                pltpu.async_copy(bufs[j], acc.at[dstr[p].at[j]],
                                 ssems[j], add=True)

            @pl.when(k + 1 < nblk)
            def _():
                wait_strip(q, k + 1)

            for j in range(NB):
                pltpu.make_async_copy(bufs[j], acc.at[dstr[p].at[j]],
                                      ssems[j]).wait()

                @pl.when(k + 1 < nblk)
                def _():
                    pltpu.async_copy(table.at[sstr[q].at[j]],
                                     bufs[j], gsems[j])

            @pl.when(k + 2 < nblk)
            def _():
                load_strip(p, k + 2)

        @pl.loop(0, nblk, step=2)
        def _(k):
            half(k, 0)
            half(k + 1, 1)

        plsc.subcore_barrier()
        pltpu.sync_copy(acc.at[pl.ds(s * RPT, RPT)],
                        out_hbm.at[c, pl.ds(s * RPT, RPT)])

    return sc_kernel


_sc_gs_cols = _make_sc_gather_scatter(True)
_sc_gs_edges = _make_sc_gather_scatter(False)


_R = 2000

_CNT_SPEC = pl.BlockSpec((NC, _R, 16), lambda i: (0, i, 0))


def _tc_matmul(x, w):
    def body(x_ref, w_ref, o_ref):
        o_ref[...] = jnp.dot(x_ref[...], w_ref[...],
                             preferred_element_type=jnp.float32)

    return pl.pallas_call(
        body,
        grid=(N // _R,),
        in_specs=[
            pl.BlockSpec((_R, x.shape[1]), lambda i: (i, 0)),
            pl.BlockSpec(w.shape, lambda i: (0, 0)),
        ],
        out_specs=pl.BlockSpec((_R, w.shape[1]), lambda i: (i, 0)),
        out_shape=jax.ShapeDtypeStruct((N, w.shape[1]), jnp.float32),
    )(x, w)


def _tc_scale(cnt, h):
    def body(cnt_ref, h_ref, g_ref, t_ref):
        dinv = lax.rsqrt(1.0 + cnt_ref[0, :, 0:1] + cnt_ref[1, :, 0:1])
        g = h_ref[...] * dinv
        g_ref[...] = g
        t_ref[0] = g[:, :D2].astype(jnp.bfloat16)
        t_ref[1] = g[:, D2:].astype(jnp.bfloat16)

    return pl.pallas_call(
        body,
        grid=(N // _R,),
        in_specs=[
            _CNT_SPEC,
            pl.BlockSpec((_R, 2 * D2), lambda i: (i, 0)),
        ],
        out_specs=[
            pl.BlockSpec((_R, 2 * D2), lambda i: (i, 0)),
            pl.BlockSpec((NC, _R, D2), lambda i: (0, i, 0)),
        ],
        out_shape=[
            jax.ShapeDtypeStruct((N, 2 * D2), jnp.float32),
            jax.ShapeDtypeStruct((NC, N, D2), jnp.bfloat16),
        ],
    )(cnt, h)


def _tc_combine12(cnt, acc, g, b, w):
    def body(cnt_ref, acc_ref, g_ref, b_ref, w_ref, o_ref, t_ref):
        dinv = lax.rsqrt(1.0 + cnt_ref[0, :, 0:1] + cnt_ref[1, :, 0:1])
        a = jnp.concatenate([acc_ref[0], acc_ref[1]], axis=1).astype(jnp.float32)
        t = (a + g_ref[...]) * dinv + jnp.reshape(b_ref[...], (1, 2 * D2))
        t = jnp.maximum(t, 0.0)
        r = jnp.dot(t, w_ref[...], preferred_element_type=jnp.float32) * dinv
        o_ref[...] = r
        t_ref[0] = r[:, :D2].astype(jnp.bfloat16)
        t_ref[1] = r[:, D2:].astype(jnp.bfloat16)

    return pl.pallas_call(
        body,
        grid=(N // _R,),
        in_specs=[
            _CNT_SPEC,
            pl.BlockSpec((NC, _R, D2), lambda i: (0, i, 0)),
            pl.BlockSpec((_R, 2 * D2), lambda i: (i, 0)),
            pl.BlockSpec((2 * D2,), lambda i: (0,)),
            pl.BlockSpec((2 * D2, 2 * D2), lambda i: (0, 0)),
        ],
        out_specs=[
            pl.BlockSpec((_R, 2 * D2), lambda i: (i, 0)),
            pl.BlockSpec((NC, _R, D2), lambda i: (0, i, 0)),
        ],
        out_shape=[
            jax.ShapeDtypeStruct((N, 2 * D2), jnp.float32),
            jax.ShapeDtypeStruct((NC, N, D2), jnp.bfloat16),
        ],
    )(cnt, acc, g, b, w)


def _tc_combine23(cnt, acc, g, b, w):
    def body(cnt_ref, acc_ref, g_ref, b_ref, w_ref, o_ref, t_ref):
        dinv = lax.rsqrt(1.0 + cnt_ref[0, :, 0:1] + cnt_ref[1, :, 0:1])
        a = jnp.concatenate([acc_ref[0], acc_ref[1]], axis=1).astype(jnp.float32)
        t = (a + g_ref[...]) * dinv + jnp.reshape(b_ref[...], (1, 2 * D2))
        t = jnp.maximum(t, 0.0)
        r = jnp.dot(t, w_ref[...], preferred_element_type=jnp.float32) * dinv
        o_ref[...] = r
        t_ref[...] = r.astype(jnp.bfloat16)

    return pl.pallas_call(
        body,
        grid=(N // _R,),
        in_specs=[
            _CNT_SPEC,
            pl.BlockSpec((NC, _R, D2), lambda i: (0, i, 0)),
            pl.BlockSpec((_R, 2 * D2), lambda i: (i, 0)),
            pl.BlockSpec((2 * D2,), lambda i: (0,)),
            pl.BlockSpec((2 * D2, D2), lambda i: (0, 0)),
        ],
        out_specs=[
            pl.BlockSpec((_R, D2), lambda i: (i, 0)),
            pl.BlockSpec((_R, D2), lambda i: (i, 0)),
        ],
        out_shape=[
            jax.ShapeDtypeStruct((N, D2), jnp.float32),
            jax.ShapeDtypeStruct((N, D2), jnp.bfloat16),
        ],
    )(cnt, acc, g, b, w)


def _tc_final(cnt, acc, g, b):
    def body(cnt_ref, acc_ref, g_ref, b_ref, o_ref):
        dinv = lax.rsqrt(1.0 + cnt_ref[0, :, 0:1] + cnt_ref[1, :, 0:1])
        a = acc_ref[0].astype(jnp.float32) + acc_ref[1].astype(jnp.float32)
        t = (a + g_ref[...]) * dinv
        o_ref[...] = t + jnp.reshape(b_ref[...], (1, D2))

    return pl.pallas_call(
        body,
        grid=(N // _R,),
        in_specs=[
            _CNT_SPEC,
            pl.BlockSpec((NC, _R, D2), lambda i: (0, i, 0)),
            pl.BlockSpec((_R, D2), lambda i: (i, 0)),
            pl.BlockSpec((D2,), lambda i: (0,)),
        ],
        out_specs=pl.BlockSpec((_R, D2), lambda i: (i, 0)),
        out_shape=jax.ShapeDtypeStruct((N, D2), jnp.float32),
    )(cnt, acc, g, b)


@jax.jit
def kernel(x, edge_index, W1, b1, W2, b2, W3, b3):
    pad = NS * EPS - E
    pad_src = jnp.arange(pad, dtype=jnp.int32) % N
    src = jnp.concatenate([edge_index[0], pad_src]).reshape(NS, CHT, B)
    dst = jnp.concatenate(
        [edge_index[1], jnp.full((pad,), N, jnp.int32)]).reshape(NS, CHT, B)

    cnt = _sc_degree(dst)

    h1 = _tc_matmul(x, W1)
    g1, g1t = _tc_scale(cnt, h1)

    acc1 = _sc_gs_cols(src, dst, g1t)
    g2, g2t = _tc_combine12(cnt, acc1, g1, b1, W2)

    acc2 = _sc_gs_cols(src, dst, g2t)
    g3, g3t = _tc_combine23(cnt, acc2, g2, b2, W3)

    acc3 = _sc_gs_edges(src, dst, g3t)
    return _tc_final(cnt, acc3, g3, b3)
